# baseline (speedup 1.0000x reference)
import functools

import jax
import jax.numpy as jnp
from jax.experimental import pallas as pl

_N = 4096
_EMB = 128
_ALPHA = 0.2
_BLK = 256


def _pass1_kernel(feature_ref, weight_ref, adj_ref, support_ref, t1_ref):
    i = pl.program_id(0)

    @pl.when(i == 0)
    def _():
        support_ref[...] = jax.nn.relu(
            jnp.dot(feature_ref[...], weight_ref[...],
                    preferred_element_type=jnp.float32))

    t1_ref[...] = jnp.dot(adj_ref[...], support_ref[...],
                          preferred_element_type=jnp.float32)


def _pass2_kernel(adj_ref, t1_ref, support_ref, cat_w_ref, bias_ref,
                  cat_b_ref, out_ref):
    i = pl.program_id(0)
    rows = pl.ds(i * _BLK, _BLK)

    t2 = jnp.dot(adj_ref[...], t1_ref[...],
                 preferred_element_type=jnp.float32)
    sup = support_ref[rows, :]
    low = t1_ref[rows, :] + sup
    mid = t2 - sup

    dims = (((1,), (1,)), ((), ()))
    lin = jax.lax.dot_general(low, cat_w_ref[:, :_EMB], dims,
                              preferred_element_type=jnp.float32)
    lin += jax.lax.dot_general(mid, cat_w_ref[:, _EMB:], dims,
                               preferred_element_type=jnp.float32)
    lin += cat_b_ref[...]
    out_ref[...] = jnp.where(lin >= 0.0, lin, _ALPHA * lin) + bias_ref[...]


@functools.partial(jax.jit, donate_argnums=())
def kernel(feature, adj, weight, bias, cat_w, cat_b):
    nblk = _N // _BLK
    bias2 = bias.reshape(1, _EMB)
    cat_b2 = cat_b.reshape(1, _EMB)

    support, t1 = pl.pallas_call(
        _pass1_kernel,
        grid=(nblk,),
        in_specs=[
            pl.BlockSpec((_N, _EMB), lambda i: (0, 0)),
            pl.BlockSpec((_EMB, _EMB), lambda i: (0, 0)),
            pl.BlockSpec((_BLK, _N), lambda i: (i, 0)),
        ],
        out_specs=[
            pl.BlockSpec((_N, _EMB), lambda i: (0, 0)),
            pl.BlockSpec((_BLK, _EMB), lambda i: (i, 0)),
        ],
        out_shape=[
            jax.ShapeDtypeStruct((_N, _EMB), jnp.float32),
            jax.ShapeDtypeStruct((_N, _EMB), jnp.float32),
        ],
    )(feature, weight, adj)

    out = pl.pallas_call(
        _pass2_kernel,
        grid=(nblk,),
        in_specs=[
            pl.BlockSpec((_BLK, _N), lambda i: (i, 0)),
            pl.BlockSpec((_N, _EMB), lambda i: (0, 0)),
            pl.BlockSpec((_N, _EMB), lambda i: (0, 0)),
            pl.BlockSpec((_EMB, 2 * _EMB), lambda i: (0, 0)),
            pl.BlockSpec((1, _EMB), lambda i: (0, 0)),
            pl.BlockSpec((1, _EMB), lambda i: (0, 0)),
        ],
        out_specs=pl.BlockSpec((_BLK, _EMB), lambda i: (i, 0)),
        out_shape=jax.ShapeDtypeStruct((_N, _EMB), jnp.float32),
    )(adj, t1, support, cat_w, bias2, cat_b2)

    return out


# speedup vs baseline: 1.1062x; 1.1062x over previous
import functools

import jax
import jax.numpy as jnp
from jax.experimental import pallas as pl

_N = 4096
_EMB = 128
_ALPHA = 0.2
_BLK = 1024


def _pass1_kernel(feature_ref, weight_ref, adj_ref, support_ref, t1_ref):
    i = pl.program_id(0)

    @pl.when(i == 0)
    def _():
        support_ref[...] = jax.nn.relu(
            jnp.dot(feature_ref[...], weight_ref[...],
                    preferred_element_type=jnp.float32))

    t1_ref[...] = jnp.dot(adj_ref[...], support_ref[...],
                          preferred_element_type=jnp.float32)


def _pass2_kernel(adj_ref, t1_ref, support_ref, cat_w_ref, bias_ref,
                  cat_b_ref, out_ref):
    i = pl.program_id(0)
    rows = pl.ds(i * _BLK, _BLK)

    t2 = jnp.dot(adj_ref[...], t1_ref[...],
                 preferred_element_type=jnp.float32)
    sup = support_ref[rows, :]
    low = t1_ref[rows, :] + sup
    mid = t2 - sup

    dims = (((1,), (1,)), ((), ()))
    lin = jax.lax.dot_general(low, cat_w_ref[:, :_EMB], dims,
                              preferred_element_type=jnp.float32)
    lin += jax.lax.dot_general(mid, cat_w_ref[:, _EMB:], dims,
                               preferred_element_type=jnp.float32)
    lin += cat_b_ref[...]
    out_ref[...] = jnp.where(lin >= 0.0, lin, _ALPHA * lin) + bias_ref[...]


@functools.partial(jax.jit, donate_argnums=())
def kernel(feature, adj, weight, bias, cat_w, cat_b):
    nblk = _N // _BLK
    bias2 = bias.reshape(1, _EMB)
    cat_b2 = cat_b.reshape(1, _EMB)

    support, t1 = pl.pallas_call(
        _pass1_kernel,
        grid=(nblk,),
        in_specs=[
            pl.BlockSpec((_N, _EMB), lambda i: (0, 0)),
            pl.BlockSpec((_EMB, _EMB), lambda i: (0, 0)),
            pl.BlockSpec((_BLK, _N), lambda i: (i, 0)),
        ],
        out_specs=[
            pl.BlockSpec((_N, _EMB), lambda i: (0, 0)),
            pl.BlockSpec((_BLK, _EMB), lambda i: (i, 0)),
        ],
        out_shape=[
            jax.ShapeDtypeStruct((_N, _EMB), jnp.float32),
            jax.ShapeDtypeStruct((_N, _EMB), jnp.float32),
        ],
    )(feature, weight, adj)

    out = pl.pallas_call(
        _pass2_kernel,
        grid=(nblk,),
        in_specs=[
            pl.BlockSpec((_BLK, _N), lambda i: (i, 0)),
            pl.BlockSpec((_N, _EMB), lambda i: (0, 0)),
            pl.BlockSpec((_N, _EMB), lambda i: (0, 0)),
            pl.BlockSpec((_EMB, 2 * _EMB), lambda i: (0, 0)),
            pl.BlockSpec((1, _EMB), lambda i: (0, 0)),
            pl.BlockSpec((1, _EMB), lambda i: (0, 0)),
        ],
        out_specs=pl.BlockSpec((_BLK, _EMB), lambda i: (i, 0)),
        out_shape=jax.ShapeDtypeStruct((_N, _EMB), jnp.float32),
    )(adj, t1, support, cat_w, bias2, cat_b2)

    return out


# speedup vs baseline: 1.4666x; 1.3258x over previous
import jax
import jax.numpy as jnp
from jax.experimental import pallas as pl
from jax.experimental.pallas import tpu as pltpu

_N = 4096
_EMB = 128
_ALPHA = 0.2
_BLK = 512
_NBLK = _N // _BLK


def _fused_kernel(feature_ref, weight_ref, adj_ref, cat_w_ref, bias_ref,
                  cat_b_ref, out_ref, support_s, t1_s, adj_bf_s):
    p = pl.program_id(0)
    i = pl.program_id(1)
    rows = pl.ds(i * _BLK, _BLK)

    @pl.when(jnp.logical_and(p == 0, i == 0))
    def _():
        support_s[...] = jax.nn.relu(
            jnp.dot(feature_ref[...], weight_ref[...],
                    preferred_element_type=jnp.float32))

    @pl.when(p == 0)
    def _():
        adj_bf_s[rows, :] = adj_ref[...].astype(jnp.bfloat16)
        t1_s[rows, :] = jnp.dot(adj_ref[...], support_s[...],
                                preferred_element_type=jnp.float32)

    @pl.when(p == 1)
    def _():
        t2 = jnp.dot(adj_bf_s[rows, :], t1_s[...],
                     preferred_element_type=jnp.float32)
        sup = support_s[rows, :]
        low = t1_s[rows, :] + sup
        mid = t2 - sup

        dims = (((1,), (1,)), ((), ()))
        lin = jax.lax.dot_general(low, cat_w_ref[:, :_EMB], dims,
                                  preferred_element_type=jnp.float32)
        lin += jax.lax.dot_general(mid, cat_w_ref[:, _EMB:], dims,
                                   preferred_element_type=jnp.float32)
        lin += cat_b_ref[...]
        out_ref[...] = jnp.where(lin >= 0.0, lin, _ALPHA * lin) + bias_ref[...]


@jax.jit
def kernel(feature, adj, weight, bias, cat_w, cat_b):
    bias2 = bias.reshape(1, _EMB)
    cat_b2 = cat_b.reshape(1, _EMB)

    out = pl.pallas_call(
        _fused_kernel,
        grid=(2, _NBLK),
        in_specs=[
            pl.BlockSpec((_N, _EMB), lambda p, i: (0, 0)),
            pl.BlockSpec((_EMB, _EMB), lambda p, i: (0, 0)),
            pl.BlockSpec((_BLK, _N),
                         lambda p, i: ((1 - p) * i + p * (_NBLK - 1), 0)),
            pl.BlockSpec((_EMB, 2 * _EMB), lambda p, i: (0, 0)),
            pl.BlockSpec((1, _EMB), lambda p, i: (0, 0)),
            pl.BlockSpec((1, _EMB), lambda p, i: (0, 0)),
        ],
        out_specs=pl.BlockSpec((_BLK, _EMB), lambda p, i: (p * i, 0)),
        out_shape=jax.ShapeDtypeStruct((_N, _EMB), jnp.float32),
        scratch_shapes=[
            pltpu.VMEM((_N, _EMB), jnp.float32),
            pltpu.VMEM((_N, _EMB), jnp.float32),
            pltpu.VMEM((_N, _N), jnp.bfloat16),
        ],
    )(feature, weight, adj, cat_w, bias2, cat_b2)

    return out
